# baseline (speedup 1.0000x reference)
import functools

import numpy as np
import jax
import jax.numpy as jnp
from jax import lax
from jax.experimental import pallas as pl
from jax.experimental.pallas import tpu as pltpu
from jax.experimental.pallas import tpu_sc as plsc

_C_IN, _C_OUT, _R_OUT, _P, _T = 4, 64, 16, 5, 16
_B = _P * _T
_N = 1024
_M = _N * _B
_K = _C_IN * _B
_G = _K * _R_OUT

_NC, _NS = 2, 16
_NW = _NC * _NS
_ROWS_PER_W = _G // _NW
_CHUNK = 80
_NCHUNK = _ROWS_PER_W // _CHUNK


def _build_gather_idx() -> np.ndarray:
    b = np.arange(_B)
    ti, pi = b // _P, b % _P
    idx = np.empty((_B, _C_IN, _R_OUT), dtype=np.int32)
    for c in range(_C_IN):
        for j in range(_R_OUT):
            idx[:, c, j] = c * _B + ((ti + j) % _T) * _P + pi
    return idx.reshape(_NW, _NCHUNK, _CHUNK)


def _build_unshuffle() -> np.ndarray:
    s = np.zeros((_C_OUT * _R_OUT, _C_OUT * _R_OUT), dtype=np.float32)
    j = np.arange(_R_OUT)[:, None]
    co = np.arange(_C_OUT)[None, :]
    s[(j * _C_OUT + co).ravel(), (co * _R_OUT + j).ravel()] = 1.0
    return s


_IDX = _build_gather_idx()
_S = _build_unshuffle()


def _sc_rotation_gather(weights, idx):
    mesh = plsc.VectorSubcoreMesh(core_axis_name="c", subcore_axis_name="s")

    @functools.partial(
        pl.kernel,
        mesh=mesh,
        out_type=jax.ShapeDtypeStruct((_G, _C_OUT), jnp.float32),
        scratch_types=[
            pltpu.VMEM((_NCHUNK, _CHUNK), jnp.int32),
            pltpu.VMEM((_NCHUNK, _CHUNK, _C_OUT), jnp.float32),
            pltpu.SemaphoreType.DMA,
        ],
        compiler_params=pltpu.CompilerParams(use_tc_tiling_on_sc=False),
    )
    def gather_kernel(w_hbm, idx_hbm, out_hbm, idx_v, rows_v, sem):
        wid = lax.axis_index("s") * _NC + lax.axis_index("c")
        pltpu.sync_copy(idx_hbm.at[wid], idx_v)
        copies = [
            pltpu.async_copy(w_hbm.at[idx_v.at[ch]], rows_v.at[ch], sem)
            for ch in range(_NCHUNK)
        ]
        for c in copies:
            c.wait()
        base = wid * _ROWS_PER_W
        for ch in range(_NCHUNK):
            pltpu.sync_copy(rows_v.at[ch], out_hbm.at[pl.ds(base + ch * _CHUNK, _CHUNK)])

    return gather_kernel(weights, idx)


_BM = 4096
_M_SC = 0
_M_TC = _M - _M_SC
_ROWS_W = _M_SC // _NW
_RG = 8
_NG = _ROWS_W // _RG
_NBUF = 2


def _conn_matvec_body(conn_ref, x_ref, *h_refs):
    blk = conn_ref[...]
    for c in range(_C_IN):
        h_refs[c][...] = jnp.sum(blk * x_ref[:, c][None, :], axis=1)


def _conn_matvec_tc(conn, x):
    return pl.pallas_call(
        _conn_matvec_body,
        grid=(_M_TC // _BM,),
        in_specs=[
            pl.BlockSpec((_BM, _N), lambda i: (i, 0)),
            pl.BlockSpec((_N, _C_IN), lambda i: (0, 0)),
        ],
        out_specs=[pl.BlockSpec((_BM,), lambda i: (i,)) for _ in range(_C_IN)],
        out_shape=[
            jax.ShapeDtypeStruct((_M_TC,), jnp.float32) for _ in range(_C_IN)
        ],
        compiler_params=pltpu.CompilerParams(
            dimension_semantics=("arbitrary",),
        ),
    )(conn, x)


def _conn_matvec_sc(conn, xt):
    mesh = plsc.VectorSubcoreMesh(core_axis_name="c", subcore_axis_name="s")

    @functools.partial(
        pl.kernel,
        mesh=mesh,
        out_type=jax.ShapeDtypeStruct((_M_SC * _C_IN,), jnp.float32),
        scratch_types=[
            pltpu.VMEM((_C_IN, _N), jnp.float32),
            pltpu.VMEM((_NBUF, _RG, _N), jnp.float32),
            pltpu.VMEM((16, 16), jnp.float32),
            pltpu.VMEM((_ROWS_W * _C_IN,), jnp.float32),
            pltpu.SemaphoreType.DMA,
            pltpu.SemaphoreType.DMA,
        ],
        compiler_params=pltpu.CompilerParams(needs_layout_passes=False),
    )
    def matvec_kernel(conn_hbm, xt_hbm, out_hbm, xt_v, buf_v, scr_v, h_v, sem0, sem1):
        wid = lax.axis_index("s") * _NC + lax.axis_index("c")
        row0 = _M_TC + wid * _ROWS_W
        pltpu.sync_copy(xt_hbm, xt_v)
        sems = [sem0, sem1]

        def _src(g):
            return conn_hbm.at[pl.ds(row0 + g * _RG, _RG)]

        for b in range(_NBUF):
            pltpu.async_copy(_src(b), buf_v.at[b], sems[b])

        lanes = lax.iota(jnp.int32, 16)

        def _compute(g, b):
            def chunk(i, accs):
                xs = [xt_v[c, pl.ds(i * 16, 16)] for c in range(_C_IN)]
                out = []
                for r in range(_RG):
                    cv = buf_v[b, r, pl.ds(i * 16, 16)]
                    for c in range(_C_IN):
                        out.append(accs[r * _C_IN + c] + cv * xs[c])
                return tuple(out)

            accs = lax.fori_loop(
                0, _N // 16, chunk,
                tuple(jnp.zeros((16,), jnp.float32) for _ in range(_RG * _C_IN)),
            )
            for half in range(2):
                for k in range(16):
                    scr_v[k, :] = accs[half * 16 + k]
                hvec = jnp.zeros((16,), jnp.float32)
                for j in range(16):
                    col = plsc.load_gather(
                        scr_v, [lanes, jnp.full((16,), j, jnp.int32)]
                    )
                    hvec = hvec + col
                h_v[pl.ds((g * 2 + half) * 16, 16)] = hvec

        def body(g0, carry):
            for b in range(_NBUF):
                g = g0 * _NBUF + b
                pltpu.make_async_copy(_src(g), buf_v.at[b], sems[b]).wait()
                _compute(g, b)

                @pl.when(g + _NBUF < _NG)
                def _():
                    pltpu.async_copy(_src(g + _NBUF), buf_v.at[b], sems[b])

            return carry

        lax.fori_loop(0, _NG // _NBUF, body, 0)
        pltpu.sync_copy(h_v, out_hbm.at[pl.ds(wid * _ROWS_W * _C_IN, _ROWS_W * _C_IN)])

    return matvec_kernel(conn, xt)


def _mix_body(hr_ref, cc_ref, s_ref, o_ref):
    lw = jnp.dot(cc_ref[...], s_ref[...], preferred_element_type=jnp.float32)
    o_ref[...] = jnp.dot(hr_ref[...], lw, preferred_element_type=jnp.float32)


def _mix(hr, cc, s):
    return pl.pallas_call(
        _mix_body,
        out_shape=jax.ShapeDtypeStruct((_N, _C_OUT * _R_OUT), jnp.float32),
    )(hr, cc, s)


def kernel(x, conn, weights):
    cg = _sc_rotation_gather(weights, jnp.asarray(_IDX))
    cc = cg.reshape(_K, _R_OUT * _C_OUT)
    h_cols = _conn_matvec_tc(conn, x)
    h = jnp.stack(h_cols, axis=1)
    if _M_SC:
        h_sc = _conn_matvec_sc(conn, x.T)
        h = jnp.concatenate([h, h_sc.reshape(_M_SC, _C_IN)], axis=0)
    hr = h.reshape(_N, _K)
    return _mix(hr, cc, jnp.asarray(_S))


# speedup vs baseline: 1.7905x; 1.7905x over previous
import functools

import numpy as np
import jax
import jax.numpy as jnp
from jax import lax
from jax.experimental import pallas as pl
from jax.experimental.pallas import tpu as pltpu
from jax.experimental.pallas import tpu_sc as plsc

_C_IN, _C_OUT, _R_OUT, _P, _T = 4, 64, 16, 5, 16
_B = _P * _T
_N = 1024
_M = _N * _B
_K = _C_IN * _B
_G = _K * _R_OUT

_NC, _NS = 2, 16
_NW = _NC * _NS
_ROWS_PER_W = _G // _NW
_CHUNK = 80
_NCHUNK = _ROWS_PER_W // _CHUNK


def _build_gather_idx() -> np.ndarray:
    b = np.arange(_B)
    ti, pi = b // _P, b % _P
    idx = np.empty((_B, _C_IN, _R_OUT), dtype=np.int32)
    for c in range(_C_IN):
        for j in range(_R_OUT):
            idx[:, c, j] = c * _B + ((ti + j) % _T) * _P + pi
    return idx.reshape(_NW, _NCHUNK, _CHUNK)


def _build_unshuffle() -> np.ndarray:
    s = np.zeros((_C_OUT * _R_OUT, _C_OUT * _R_OUT), dtype=np.float32)
    j = np.arange(_R_OUT)[:, None]
    co = np.arange(_C_OUT)[None, :]
    s[(j * _C_OUT + co).ravel(), (co * _R_OUT + j).ravel()] = 1.0
    return s


_IDX = _build_gather_idx()
_S = _build_unshuffle()


def _sc_rotation_gather(weights, idx):
    mesh = plsc.VectorSubcoreMesh(core_axis_name="c", subcore_axis_name="s")

    @functools.partial(
        pl.kernel,
        mesh=mesh,
        out_type=jax.ShapeDtypeStruct((_G, _C_OUT), jnp.float32),
        scratch_types=[
            pltpu.VMEM((_NCHUNK, _CHUNK), jnp.int32),
            pltpu.VMEM((_NCHUNK, _CHUNK, _C_OUT), jnp.float32),
            pltpu.SemaphoreType.DMA,
        ],
        compiler_params=pltpu.CompilerParams(use_tc_tiling_on_sc=False),
    )
    def gather_kernel(w_hbm, idx_hbm, out_hbm, idx_v, rows_v, sem):
        wid = lax.axis_index("s") * _NC + lax.axis_index("c")
        pltpu.sync_copy(idx_hbm.at[wid], idx_v)
        copies = [
            pltpu.async_copy(w_hbm.at[idx_v.at[ch]], rows_v.at[ch], sem)
            for ch in range(_NCHUNK)
        ]
        for c in copies:
            c.wait()
        base = wid * _ROWS_PER_W
        for ch in range(_NCHUNK):
            pltpu.sync_copy(rows_v.at[ch], out_hbm.at[pl.ds(base + ch * _CHUNK, _CHUNK)])

    return gather_kernel(weights, idx)


_BM = 5120
_M_SC = 0
_M_TC = _M - _M_SC
_ROWS_W = _M_SC // _NW
_RG = 8
_NG = _ROWS_W // _RG
_NBUF = 2


def _conn_matvec_body(conn_ref, x_ref, h_ref):
    h_ref[...] = jnp.dot(conn_ref[...], x_ref[...], preferred_element_type=jnp.float32)


def _conn_matvec_tc(conn, x):
    return pl.pallas_call(
        _conn_matvec_body,
        grid=(_M // _BM,),
        in_specs=[
            pl.BlockSpec((_BM, _N), lambda i: (i, 0)),
            pl.BlockSpec((_N, _C_IN), lambda i: (0, 0)),
        ],
        out_specs=pl.BlockSpec((_BM, _C_IN), lambda i: (i, 0)),
        out_shape=jax.ShapeDtypeStruct((_M, _C_IN), jnp.float32),
        compiler_params=pltpu.CompilerParams(
            dimension_semantics=("arbitrary",),
            vmem_limit_bytes=100 * 1024 * 1024,
        ),
    )(conn, x)


def _conn_matvec_sc(conn, xt):
    mesh = plsc.VectorSubcoreMesh(core_axis_name="c", subcore_axis_name="s")

    @functools.partial(
        pl.kernel,
        mesh=mesh,
        out_type=jax.ShapeDtypeStruct((_M_SC * _C_IN,), jnp.float32),
        scratch_types=[
            pltpu.VMEM((_C_IN, _N), jnp.float32),
            pltpu.VMEM((_NBUF, _RG, _N), jnp.float32),
            pltpu.VMEM((16, 16), jnp.float32),
            pltpu.VMEM((_ROWS_W * _C_IN,), jnp.float32),
            pltpu.SemaphoreType.DMA,
            pltpu.SemaphoreType.DMA,
        ],
        compiler_params=pltpu.CompilerParams(needs_layout_passes=False),
    )
    def matvec_kernel(conn_hbm, xt_hbm, out_hbm, xt_v, buf_v, scr_v, h_v, sem0, sem1):
        wid = lax.axis_index("s") * _NC + lax.axis_index("c")
        row0 = _M_TC + wid * _ROWS_W
        pltpu.sync_copy(xt_hbm, xt_v)
        sems = [sem0, sem1]

        def _src(g):
            return conn_hbm.at[pl.ds(row0 + g * _RG, _RG)]

        for b in range(_NBUF):
            pltpu.async_copy(_src(b), buf_v.at[b], sems[b])

        lanes = lax.iota(jnp.int32, 16)

        def _compute(g, b):
            def chunk(i, accs):
                xs = [xt_v[c, pl.ds(i * 16, 16)] for c in range(_C_IN)]
                out = []
                for r in range(_RG):
                    cv = buf_v[b, r, pl.ds(i * 16, 16)]
                    for c in range(_C_IN):
                        out.append(accs[r * _C_IN + c] + cv * xs[c])
                return tuple(out)

            accs = lax.fori_loop(
                0, _N // 16, chunk,
                tuple(jnp.zeros((16,), jnp.float32) for _ in range(_RG * _C_IN)),
            )
            for half in range(2):
                for k in range(16):
                    scr_v[k, :] = accs[half * 16 + k]
                hvec = jnp.zeros((16,), jnp.float32)
                for j in range(16):
                    col = plsc.load_gather(
                        scr_v, [lanes, jnp.full((16,), j, jnp.int32)]
                    )
                    hvec = hvec + col
                h_v[pl.ds((g * 2 + half) * 16, 16)] = hvec

        def body(g0, carry):
            for b in range(_NBUF):
                g = g0 * _NBUF + b
                pltpu.make_async_copy(_src(g), buf_v.at[b], sems[b]).wait()
                _compute(g, b)

                @pl.when(g + _NBUF < _NG)
                def _():
                    pltpu.async_copy(_src(g + _NBUF), buf_v.at[b], sems[b])

            return carry

        lax.fori_loop(0, _NG // _NBUF, body, 0)
        pltpu.sync_copy(h_v, out_hbm.at[pl.ds(wid * _ROWS_W * _C_IN, _ROWS_W * _C_IN)])

    return matvec_kernel(conn, xt)


def _mix_body(hr_ref, cc_ref, s_ref, o_ref):
    lw = jnp.dot(cc_ref[...], s_ref[...], preferred_element_type=jnp.float32)
    o_ref[...] = jnp.dot(hr_ref[...], lw, preferred_element_type=jnp.float32)


def _mix(hr, cc, s):
    return pl.pallas_call(
        _mix_body,
        out_shape=jax.ShapeDtypeStruct((_N, _C_OUT * _R_OUT), jnp.float32),
    )(hr, cc, s)


def kernel(x, conn, weights):
    cg = _sc_rotation_gather(weights, jnp.asarray(_IDX))
    cc = cg.reshape(_K, _R_OUT * _C_OUT)
    h = _conn_matvec_tc(conn, x)
    hr = h.reshape(_N, _K)
    return _mix(hr, cc, jnp.asarray(_S))


# speedup vs baseline: 1.8091x; 1.0104x over previous
import functools

import numpy as np
import jax
import jax.numpy as jnp
from jax import lax
from jax.experimental import pallas as pl
from jax.experimental.pallas import tpu as pltpu
from jax.experimental.pallas import tpu_sc as plsc

_C_IN, _C_OUT, _R_OUT, _P, _T = 4, 64, 16, 5, 16
_B = _P * _T
_N = 1024
_M = _N * _B
_K = _C_IN * _B
_G = _K * _R_OUT

_NC, _NS = 2, 16
_NW = _NC * _NS
_ROWS_PER_W = _G // _NW
_CHUNK = 80
_NCHUNK = _ROWS_PER_W // _CHUNK


def _build_gather_idx() -> np.ndarray:
    b = np.arange(_B)
    ti, pi = b // _P, b % _P
    idx = np.empty((_B, _C_IN, _R_OUT), dtype=np.int32)
    for c in range(_C_IN):
        for j in range(_R_OUT):
            idx[:, c, j] = c * _B + ((ti + j) % _T) * _P + pi
    return idx.reshape(_NW, _NCHUNK, _CHUNK)


def _build_unshuffle() -> np.ndarray:
    s = np.zeros((_C_OUT * _R_OUT, _C_OUT * _R_OUT), dtype=np.float32)
    j = np.arange(_R_OUT)[:, None]
    co = np.arange(_C_OUT)[None, :]
    s[(j * _C_OUT + co).ravel(), (co * _R_OUT + j).ravel()] = 1.0
    return s


_IDX = _build_gather_idx()
_S = _build_unshuffle()


def _sc_rotation_gather(weights, idx):
    mesh = plsc.VectorSubcoreMesh(core_axis_name="c", subcore_axis_name="s")

    @functools.partial(
        pl.kernel,
        mesh=mesh,
        out_type=jax.ShapeDtypeStruct((_G, _C_OUT), jnp.float32),
        scratch_types=[
            pltpu.VMEM((_NCHUNK, _CHUNK), jnp.int32),
            pltpu.VMEM((_NCHUNK, _CHUNK, _C_OUT), jnp.float32),
            pltpu.SemaphoreType.DMA,
        ],
        compiler_params=pltpu.CompilerParams(use_tc_tiling_on_sc=False),
    )
    def gather_kernel(w_hbm, idx_hbm, out_hbm, idx_v, rows_v, sem):
        wid = lax.axis_index("s") * _NC + lax.axis_index("c")
        pltpu.sync_copy(idx_hbm.at[wid], idx_v)
        copies = [
            pltpu.async_copy(w_hbm.at[idx_v.at[ch]], rows_v.at[ch], sem)
            for ch in range(_NCHUNK)
        ]
        for c in copies:
            c.wait()
        base = wid * _ROWS_PER_W
        for ch in range(_NCHUNK):
            pltpu.sync_copy(rows_v.at[ch], out_hbm.at[pl.ds(base + ch * _CHUNK, _CHUNK)])

    return gather_kernel(weights, idx)


_BM = 4096
_M_SC = 0
_M_TC = _M - _M_SC
_ROWS_W = _M_SC // _NW
_RG = 8
_NG = _ROWS_W // _RG
_NBUF = 2


def _conn_matvec_body(conn_ref, x_ref, h_ref):
    h_ref[...] = jnp.dot(conn_ref[...], x_ref[...], preferred_element_type=jnp.float32)


def _conn_matvec_tc(conn, x):
    return pl.pallas_call(
        _conn_matvec_body,
        grid=(_M // _BM,),
        in_specs=[
            pl.BlockSpec((_BM, _N), lambda i: (i, 0)),
            pl.BlockSpec((_N, _C_IN), lambda i: (0, 0)),
        ],
        out_specs=pl.BlockSpec((_BM, _C_IN), lambda i: (i, 0)),
        out_shape=jax.ShapeDtypeStruct((_M, _C_IN), jnp.float32),
        compiler_params=pltpu.CompilerParams(
            dimension_semantics=("arbitrary",),
            vmem_limit_bytes=100 * 1024 * 1024,
        ),
    )(conn, x)


def _conn_matvec_sc(conn, xt):
    mesh = plsc.VectorSubcoreMesh(core_axis_name="c", subcore_axis_name="s")

    @functools.partial(
        pl.kernel,
        mesh=mesh,
        out_type=jax.ShapeDtypeStruct((_M_SC * _C_IN,), jnp.float32),
        scratch_types=[
            pltpu.VMEM((_C_IN, _N), jnp.float32),
            pltpu.VMEM((_NBUF, _RG, _N), jnp.float32),
            pltpu.VMEM((16, 16), jnp.float32),
            pltpu.VMEM((_ROWS_W * _C_IN,), jnp.float32),
            pltpu.SemaphoreType.DMA,
            pltpu.SemaphoreType.DMA,
        ],
        compiler_params=pltpu.CompilerParams(needs_layout_passes=False),
    )
    def matvec_kernel(conn_hbm, xt_hbm, out_hbm, xt_v, buf_v, scr_v, h_v, sem0, sem1):
        wid = lax.axis_index("s") * _NC + lax.axis_index("c")
        row0 = _M_TC + wid * _ROWS_W
        pltpu.sync_copy(xt_hbm, xt_v)
        sems = [sem0, sem1]

        def _src(g):
            return conn_hbm.at[pl.ds(row0 + g * _RG, _RG)]

        for b in range(_NBUF):
            pltpu.async_copy(_src(b), buf_v.at[b], sems[b])

        lanes = lax.iota(jnp.int32, 16)

        def _compute(g, b):
            def chunk(i, accs):
                xs = [xt_v[c, pl.ds(i * 16, 16)] for c in range(_C_IN)]
                out = []
                for r in range(_RG):
                    cv = buf_v[b, r, pl.ds(i * 16, 16)]
                    for c in range(_C_IN):
                        out.append(accs[r * _C_IN + c] + cv * xs[c])
                return tuple(out)

            accs = lax.fori_loop(
                0, _N // 16, chunk,
                tuple(jnp.zeros((16,), jnp.float32) for _ in range(_RG * _C_IN)),
            )
            for half in range(2):
                for k in range(16):
                    scr_v[k, :] = accs[half * 16 + k]
                hvec = jnp.zeros((16,), jnp.float32)
                for j in range(16):
                    col = plsc.load_gather(
                        scr_v, [lanes, jnp.full((16,), j, jnp.int32)]
                    )
                    hvec = hvec + col
                h_v[pl.ds((g * 2 + half) * 16, 16)] = hvec

        def body(g0, carry):
            for b in range(_NBUF):
                g = g0 * _NBUF + b
                pltpu.make_async_copy(_src(g), buf_v.at[b], sems[b]).wait()
                _compute(g, b)

                @pl.when(g + _NBUF < _NG)
                def _():
                    pltpu.async_copy(_src(g + _NBUF), buf_v.at[b], sems[b])

            return carry

        lax.fori_loop(0, _NG // _NBUF, body, 0)
        pltpu.sync_copy(h_v, out_hbm.at[pl.ds(wid * _ROWS_W * _C_IN, _ROWS_W * _C_IN)])

    return matvec_kernel(conn, xt)


_KC = _C_OUT * _R_OUT


def _mix_body(hr_ref, cc_ref, o_ref):
    p = lax.broadcasted_iota(jnp.int32, (_KC, _KC), 0)
    q = lax.broadcasted_iota(jnp.int32, (_KC, _KC), 1)
    s = jnp.where(q == (p % _C_OUT) * _R_OUT + p // _C_OUT, 1.0, 0.0)
    lw = jnp.dot(cc_ref[...], s, preferred_element_type=jnp.float32)
    o_ref[...] = jnp.dot(hr_ref[...], lw, preferred_element_type=jnp.float32)


def _mix(hr, cc):
    return pl.pallas_call(
        _mix_body,
        out_shape=jax.ShapeDtypeStruct((_N, _KC), jnp.float32),
    )(hr, cc)


def kernel(x, conn, weights):
    cg = _sc_rotation_gather(weights, jnp.asarray(_IDX))
    cc = cg.reshape(_K, _R_OUT * _C_OUT)
    h = _conn_matvec_tc(conn, x)
    hr = h.reshape(_N, _K)
    return _mix(hr, cc)


# speedup vs baseline: 1.8109x; 1.0010x over previous
import functools

import numpy as np
import jax
import jax.numpy as jnp
from jax import lax
from jax.experimental import pallas as pl
from jax.experimental.pallas import tpu as pltpu
from jax.experimental.pallas import tpu_sc as plsc

_C_IN, _C_OUT, _R_OUT, _P, _T = 4, 64, 16, 5, 16
_B = _P * _T
_N = 1024
_M = _N * _B
_K = _C_IN * _B
_G = _K * _R_OUT

_NC, _NS = 2, 16
_NW = _NC * _NS
_ROWS_PER_W = _G // _NW
_CHUNK = 80
_NCHUNK = _ROWS_PER_W // _CHUNK


def _build_gather_idx() -> np.ndarray:
    b = np.arange(_B)
    ti, pi = b // _P, b % _P
    idx = np.empty((_B, _C_IN, _R_OUT), dtype=np.int32)
    for c in range(_C_IN):
        for j in range(_R_OUT):
            idx[:, c, j] = c * _B + ((ti + j) % _T) * _P + pi
    return idx.reshape(_NW, _NCHUNK, _CHUNK)


def _build_unshuffle() -> np.ndarray:
    s = np.zeros((_C_OUT * _R_OUT, _C_OUT * _R_OUT), dtype=np.float32)
    j = np.arange(_R_OUT)[:, None]
    co = np.arange(_C_OUT)[None, :]
    s[(j * _C_OUT + co).ravel(), (co * _R_OUT + j).ravel()] = 1.0
    return s


_IDX = _build_gather_idx()
_S = _build_unshuffle()


def _sc_rotation_gather(weights, idx):
    mesh = plsc.VectorSubcoreMesh(core_axis_name="c", subcore_axis_name="s")

    @functools.partial(
        pl.kernel,
        mesh=mesh,
        out_type=jax.ShapeDtypeStruct((_G, _C_OUT), jnp.float32),
        scratch_types=[
            pltpu.VMEM((_NCHUNK, _CHUNK), jnp.int32),
            pltpu.VMEM((_NCHUNK, _CHUNK, _C_OUT), jnp.float32),
            pltpu.SemaphoreType.DMA,
        ],
        compiler_params=pltpu.CompilerParams(use_tc_tiling_on_sc=False),
    )
    def gather_kernel(w_hbm, idx_hbm, out_hbm, idx_v, rows_v, sem):
        wid = lax.axis_index("s") * _NC + lax.axis_index("c")
        pltpu.sync_copy(idx_hbm.at[wid], idx_v)
        copies = [
            pltpu.async_copy(w_hbm.at[idx_v.at[ch]], rows_v.at[ch], sem)
            for ch in range(_NCHUNK)
        ]
        for c in copies:
            c.wait()
        base = wid * _ROWS_PER_W
        for ch in range(_NCHUNK):
            pltpu.sync_copy(rows_v.at[ch], out_hbm.at[pl.ds(base + ch * _CHUNK, _CHUNK)])

    return gather_kernel(weights, idx)


_BM = 4096
_M_SC = 0
_M_TC = _M - _M_SC
_ROWS_W = _M_SC // _NW
_RG = 8
_NG = _ROWS_W // _RG
_NBUF = 2


def _conn_matvec_body(conn_ref, x_ref, h_ref):
    h_ref[...] = jnp.dot(conn_ref[...], x_ref[...], preferred_element_type=jnp.float32)


def _conn_matvec_tc(conn, x):
    return pl.pallas_call(
        _conn_matvec_body,
        grid=(_M // _BM,),
        in_specs=[
            pl.BlockSpec((_BM, _N), lambda i: (i, 0)),
            pl.BlockSpec((_N, _C_IN), lambda i: (0, 0)),
        ],
        out_specs=pl.BlockSpec((_BM, _C_IN), lambda i: (i, 0)),
        out_shape=jax.ShapeDtypeStruct((_M, _C_IN), jnp.float32),
        compiler_params=pltpu.CompilerParams(
            dimension_semantics=("arbitrary",),
            vmem_limit_bytes=100 * 1024 * 1024,
        ),
    )(conn, x)


def _conn_matvec_sc(conn, xt):
    mesh = plsc.VectorSubcoreMesh(core_axis_name="c", subcore_axis_name="s")

    @functools.partial(
        pl.kernel,
        mesh=mesh,
        out_type=jax.ShapeDtypeStruct((_M_SC * _C_IN,), jnp.float32),
        scratch_types=[
            pltpu.VMEM((_C_IN, _N), jnp.float32),
            pltpu.VMEM((_NBUF, _RG, _N), jnp.float32),
            pltpu.VMEM((16, 16), jnp.float32),
            pltpu.VMEM((_ROWS_W * _C_IN,), jnp.float32),
            pltpu.SemaphoreType.DMA,
            pltpu.SemaphoreType.DMA,
        ],
        compiler_params=pltpu.CompilerParams(needs_layout_passes=False),
    )
    def matvec_kernel(conn_hbm, xt_hbm, out_hbm, xt_v, buf_v, scr_v, h_v, sem0, sem1):
        wid = lax.axis_index("s") * _NC + lax.axis_index("c")
        row0 = _M_TC + wid * _ROWS_W
        pltpu.sync_copy(xt_hbm, xt_v)
        sems = [sem0, sem1]

        def _src(g):
            return conn_hbm.at[pl.ds(row0 + g * _RG, _RG)]

        for b in range(_NBUF):
            pltpu.async_copy(_src(b), buf_v.at[b], sems[b])

        lanes = lax.iota(jnp.int32, 16)

        def _compute(g, b):
            def chunk(i, accs):
                xs = [xt_v[c, pl.ds(i * 16, 16)] for c in range(_C_IN)]
                out = []
                for r in range(_RG):
                    cv = buf_v[b, r, pl.ds(i * 16, 16)]
                    for c in range(_C_IN):
                        out.append(accs[r * _C_IN + c] + cv * xs[c])
                return tuple(out)

            accs = lax.fori_loop(
                0, _N // 16, chunk,
                tuple(jnp.zeros((16,), jnp.float32) for _ in range(_RG * _C_IN)),
            )
            for half in range(2):
                for k in range(16):
                    scr_v[k, :] = accs[half * 16 + k]
                hvec = jnp.zeros((16,), jnp.float32)
                for j in range(16):
                    col = plsc.load_gather(
                        scr_v, [lanes, jnp.full((16,), j, jnp.int32)]
                    )
                    hvec = hvec + col
                h_v[pl.ds((g * 2 + half) * 16, 16)] = hvec

        def body(g0, carry):
            for b in range(_NBUF):
                g = g0 * _NBUF + b
                pltpu.make_async_copy(_src(g), buf_v.at[b], sems[b]).wait()
                _compute(g, b)

                @pl.when(g + _NBUF < _NG)
                def _():
                    pltpu.async_copy(_src(g + _NBUF), buf_v.at[b], sems[b])

            return carry

        lax.fori_loop(0, _NG // _NBUF, body, 0)
        pltpu.sync_copy(h_v, out_hbm.at[pl.ds(wid * _ROWS_W * _C_IN, _ROWS_W * _C_IN)])

    return matvec_kernel(conn, xt)


_KC = _C_OUT * _R_OUT


def _mix_body(hr_ref, cc_ref, o_ref):
    p = lax.broadcasted_iota(jnp.int32, (_KC, _KC), 0)
    q = lax.broadcasted_iota(jnp.int32, (_KC, _KC), 1)
    s = jnp.where(q == (p % _C_OUT) * _R_OUT + p // _C_OUT, 1.0, 0.0).astype(jnp.bfloat16)
    lw = jnp.dot(cc_ref[...].astype(jnp.bfloat16), s, preferred_element_type=jnp.float32)
    o_ref[...] = jnp.dot(
        hr_ref[...].astype(jnp.bfloat16),
        lw.astype(jnp.bfloat16),
        preferred_element_type=jnp.float32,
    )


def _mix(hr, cc):
    return pl.pallas_call(
        _mix_body,
        out_shape=jax.ShapeDtypeStruct((_N, _KC), jnp.float32),
    )(hr, cc)


def kernel(x, conn, weights):
    cg = _sc_rotation_gather(weights, jnp.asarray(_IDX))
    cc = cg.reshape(_K, _R_OUT * _C_OUT)
    h = _conn_matvec_tc(conn, x)
    hr = h.reshape(_N, _K)
    return _mix(hr, cc)
